# Initial kernel scaffold; baseline (speedup 1.0000x reference)
import jax
import jax.numpy as jnp
from jax.experimental import pallas as pl


def kernel(X, Y, S, A):
    raise NotImplementedError("write your pallas kernel here")


# speedup vs baseline: 260.6690x; 260.6690x over previous
import functools

import jax
import jax.numpy as jnp
from jax.experimental import pallas as pl

N = 1024
M_DIM = 256
S_SPARSE = 32
BATCH = 64
CG_ITERS = 32
EPS = 1e-6


def _extract_topk_mask(vals, lane, k):
    def step(_, m):
        cur = jnp.where(m > 0.0, -1.0, vals)
        mx = jnp.max(cur, axis=1, keepdims=True)
        eq = cur == mx
        idx = jnp.min(jnp.where(eq, lane, N + N), axis=1, keepdims=True)
        return m + (lane == idx).astype(jnp.float32)

    return jax.lax.fori_loop(0, k, step, jnp.zeros_like(vals))


def _cosamp_body(xt_ref, y_ref, st_ref, a_ref, at_ref, h_ref, k_ref):
    xt = xt_ref[...]
    y = y_ref[...]
    st = st_ref[...]
    a = a_ref[...]
    at = at_ref[...]

    r2 = y - jnp.dot(xt, at, preferred_element_type=jnp.float32)
    p2 = jnp.dot(r2, a, preferred_element_type=jnp.float32)

    lane = jax.lax.broadcasted_iota(jnp.int32, (BATCH, N), 1)

    smask = jnp.maximum(_extract_topk_mask(jnp.abs(p2), lane, 2 * S_SPARSE), st)

    aty = jnp.dot(y, a, preferred_element_type=jnp.float32)
    rhs = smask * aty

    def cg_step(_, carry):
        w, r, p, rs = carry
        ap = jnp.dot(p, at, preferred_element_type=jnp.float32)
        q = smask * jnp.dot(ap, a, preferred_element_type=jnp.float32) + EPS * p
        pq = jnp.sum(p * q, axis=1, keepdims=True)
        alpha = rs / jnp.maximum(pq, 1e-30)
        w = w + alpha * p
        r = r - alpha * q
        rs_new = jnp.sum(r * r, axis=1, keepdims=True)
        beta = rs_new / jnp.maximum(rs, 1e-30)
        p = r + beta * p
        return w, r, p, rs_new

    w0 = jnp.zeros_like(rhs)
    rs0 = jnp.sum(rhs * rhs, axis=1, keepdims=True)
    w, _, _, _ = jax.lax.fori_loop(0, CG_ITERS, cg_step, (w0, rhs, rhs, rs0))

    keep = _extract_topk_mask(jnp.abs(w), lane, S_SPARSE)
    h_ref[...] = w * keep
    k_ref[...] = keep


@jax.jit
def kernel(X, Y, S, A):
    xt = X.T.astype(jnp.float32)
    st = S.T.astype(jnp.float32)
    at = A.T
    h2, k2 = pl.pallas_call(
        _cosamp_body,
        out_shape=(
            jax.ShapeDtypeStruct((BATCH, N), jnp.float32),
            jax.ShapeDtypeStruct((BATCH, N), jnp.float32),
        ),
    )(xt, Y, st, A, at)
    return h2.T, (k2 > 0.5).T


# speedup vs baseline: 425.5713x; 1.6326x over previous
import functools

import jax
import jax.numpy as jnp
from jax.experimental import pallas as pl

N = 1024
M_DIM = 256
S_SPARSE = 32
BATCH = 64
CG_ITERS = 24
EPS = 1e-6


def _extract_topk_mask(vals, lane, k):
    bits = jax.lax.bitcast_convert_type(vals, jnp.int32)

    def val_step(_, c):
        lo, hi = c
        mid = lo + jax.lax.shift_right_logical(hi - lo, 1)
        cnt = jnp.sum((bits > mid).astype(jnp.int32), axis=1, keepdims=True)
        big = cnt >= k
        return jnp.where(big, mid, lo), jnp.where(big, hi, mid)

    lo0 = jnp.full((BATCH, 1), -1, jnp.int32)
    hi0 = jnp.full((BATCH, 1), 0x7F800000, jnp.int32)
    lo, hi = jax.lax.fori_loop(0, 31, val_step, (lo0, hi0))
    gt = bits > hi
    eq = bits == hi
    need = k - jnp.sum(gt.astype(jnp.int32), axis=1, keepdims=True)

    def idx_step(_, c):
        lo2, hi2 = c
        mid = jax.lax.shift_right_logical(lo2 + hi2, 1)
        cnt = jnp.sum((eq & (lane < mid)).astype(jnp.int32), axis=1,
                      keepdims=True)
        enough = cnt >= need
        return jnp.where(enough, lo2, mid), jnp.where(enough, mid, hi2)

    z = jnp.zeros((BATCH, 1), jnp.int32)
    _, cut = jax.lax.fori_loop(0, 10, idx_step, (z, z + N))
    return (gt | (eq & (lane < cut))).astype(jnp.float32)


def _cosamp_body(xt_ref, y_ref, st_ref, a_ref, at_ref, h_ref, k_ref):
    xt = xt_ref[...]
    y = y_ref[...]
    st = st_ref[...]
    a = a_ref[...]
    at = at_ref[...]

    r2 = y - jnp.dot(xt, at, preferred_element_type=jnp.float32)
    p2 = jnp.dot(r2, a, preferred_element_type=jnp.float32)

    lane = jax.lax.broadcasted_iota(jnp.int32, (BATCH, N), 1)

    smask = jnp.maximum(_extract_topk_mask(jnp.abs(p2), lane, 2 * S_SPARSE), st)

    aty = jnp.dot(y, a, preferred_element_type=jnp.float32)
    rhs = smask * aty

    def cg_step(_, carry):
        w, r, p, rs = carry
        ap = jnp.dot(p, at, preferred_element_type=jnp.float32)
        q = smask * jnp.dot(ap, a, preferred_element_type=jnp.float32) + EPS * p
        pq = jnp.sum(p * q, axis=1, keepdims=True)
        alpha = rs / jnp.maximum(pq, 1e-30)
        w = w + alpha * p
        r = r - alpha * q
        rs_new = jnp.sum(r * r, axis=1, keepdims=True)
        beta = rs_new / jnp.maximum(rs, 1e-30)
        p = r + beta * p
        return w, r, p, rs_new

    w0 = jnp.zeros_like(rhs)
    rs0 = jnp.sum(rhs * rhs, axis=1, keepdims=True)
    w, _, _, _ = jax.lax.fori_loop(0, CG_ITERS, cg_step, (w0, rhs, rhs, rs0))

    keep = _extract_topk_mask(jnp.abs(w), lane, S_SPARSE)
    h_ref[...] = w * keep
    k_ref[...] = keep


@jax.jit
def kernel(X, Y, S, A):
    xt = X.T.astype(jnp.float32)
    st = S.T.astype(jnp.float32)
    at = A.T
    h2, k2 = pl.pallas_call(
        _cosamp_body,
        out_shape=(
            jax.ShapeDtypeStruct((BATCH, N), jnp.float32),
            jax.ShapeDtypeStruct((BATCH, N), jnp.float32),
        ),
    )(xt, Y, st, A, at)
    return h2.T, (k2 > 0.5).T
